# baseline (device time: 63392 ns/iter reference)
import jax
import jax.numpy as jnp
from jax import lax
from jax.experimental import pallas as pl
from jax.experimental.pallas import tpu as pltpu

N_DEV = 16
MASKS = [1, 2, 4, 8]


def kernel(x, Wq, K_ext, V_ext, Wo):
    B, Sq, D = x.shape
    _, Hd = Wq.shape
    _, Skv, Hq, Dh = K_ext.shape
    Hloc = Hd // Dh
    T = B * Sq

    def body(x_ref, wq_ref, kext_ref, vext_ref, wo_ref, acc_ref,
             k_ref, v_ref,
             sb0, sb1, xs0, xs1, as0, as1,
             rb0, rb1, xr0, xr1, ar0, ar1,
             kv_sems, send_sems, recv_sems):
        my_pos = lax.axis_index("i")
        pending = []

        kv_copies = []
        for b in range(B):
            for h in range(Hloc):
                gh = my_pos * Hloc + h
                for i, (src, dst) in enumerate(
                        [(kext_ref, k_ref), (vext_ref, v_ref)]):
                    cp = pltpu.make_async_copy(
                        src.at[b, :, gh, :], dst.at[b, h],
                        kv_sems.at[i * B * Hloc + b * Hloc + h])
                    cp.start()
                    kv_copies.append(cp)

        barrier_sem = pltpu.get_barrier_semaphore()
        for m in MASKS:
            pl.semaphore_signal(
                barrier_sem, inc=1,
                device_id=(jnp.bitwise_xor(my_pos, m),),
                device_id_type=pl.DeviceIdType.MESH)

        qi = lax.broadcasted_iota(jnp.int32, (Sq, Skv), 0)
        ki = lax.broadcasted_iota(jnp.int32, (Sq, Skv), 1)
        mask = (jnp.abs(qi - ki) <= 128) | (ki < 32) | (qi < 32)

        kv_waited = []

        def slab(b):
            qfull = jnp.dot(x_ref[b], wq_ref[...],
                            preferred_element_type=jnp.float32)
            if not kv_waited:
                for cp in kv_copies:
                    cp.wait()
                kv_waited.append(True)
            acc = jnp.zeros((Sq, D), jnp.float32)
            for h in range(Hloc):
                q = qfull[:, h * Dh:(h + 1) * Dh]
                k = k_ref[b, h]
                v = v_ref[b, h]
                s = lax.dot_general(
                    q, k, (((1,), (1,)), ((), ())),
                    preferred_element_type=jnp.float32) * 0.125
                s = jnp.where(mask, s, -1e9)
                mx = jnp.max(s, axis=1, keepdims=True)
                w = jnp.exp(s - mx)
                w = w / jnp.sum(w, axis=1, keepdims=True)
                ctx = jnp.dot(w, v, preferred_element_type=jnp.float32)
                acc = acc + jnp.dot(
                    ctx, wo_ref[h * Dh:(h + 1) * Dh, :],
                    preferred_element_type=jnp.float32)
            acc_ref[b * Sq:(b + 1) * Sq, :] = acc

        def exchange(src_buf, dst_buf, m, slot):
            rdma = pltpu.make_async_remote_copy(
                src_ref=src_buf, dst_ref=dst_buf,
                send_sem=send_sems.at[slot], recv_sem=recv_sems.at[slot],
                device_id=(jnp.bitwise_xor(my_pos, m),),
                device_id_type=pl.DeviceIdType.MESH,
            )
            pending.append(rdma)
            return rdma

        hl = T // 2
        bit0 = (jnp.bitwise_and(my_pos, 1) > 0).astype(jnp.int32)
        rdma0 = exchange(sb0, rb0, 1, 0)

        slab(0)
        pl.semaphore_wait(barrier_sem, len(MASKS))

        @pl.when(bit0 == 1)
        def _():
            sb0[...] = acc_ref[0:hl, :].astype(jnp.bfloat16)
            rdma0.start()

        slab(1)

        @pl.when(bit0 == 0)
        def _():
            sb0[...] = acc_ref[hl:T, :].astype(jnp.bfloat16)
            rdma0.start()

        rdma0.wait_recv()
        o = pl.multiple_of(hl * bit0, hl)
        acc_ref[pl.ds(o, hl), :] = (
            acc_ref[pl.ds(o, hl), :] + rb0[...].astype(jnp.float32))

        hl = T // 4
        bit = (jnp.bitwise_and(my_pos, 4) > 0).astype(jnp.int32)
        send_off = pl.multiple_of(o + hl * (1 - bit), hl)
        keep_off = pl.multiple_of(o + hl * bit, hl)
        sb1[...] = acc_ref[pl.ds(send_off, hl), :].astype(jnp.bfloat16)
        rdma = exchange(sb1, rb1, 4, 1)
        rdma.start()
        rdma.wait_recv()
        acc_ref[pl.ds(keep_off, hl), :] = (
            acc_ref[pl.ds(keep_off, hl), :] + rb1[...].astype(jnp.float32))
        o = keep_off

        for slot, (m, sbuf, rbuf) in enumerate(
                [(2, xs0, xr0), (8, xs1, xr1)], start=2):
            sbuf[...] = acc_ref[pl.ds(o, hl), :].astype(jnp.bfloat16)
            rdma = exchange(sbuf, rbuf, m, slot)
            rdma.start()
            rdma.wait_recv()
            acc_ref[pl.ds(o, hl), :] = (
                acc_ref[pl.ds(o, hl), :] + rbuf[...].astype(jnp.float32))

        for slot, (m, bl, sbuf, rbuf) in enumerate(
                [(4, T // 4, as0, ar0), (1, T // 2, as1, ar1)], start=4):
            bit = (jnp.bitwise_and(my_pos, m) > 0).astype(jnp.int32)
            sbuf[...] = acc_ref[pl.ds(o, bl), :].astype(jnp.bfloat16)
            rdma = exchange(sbuf, rbuf, m, slot)
            rdma.start()
            rdma.wait_recv()
            recv_off = pl.multiple_of(o + bl * (1 - 2 * bit), bl)
            acc_ref[pl.ds(recv_off, bl), :] = rbuf[...].astype(jnp.float32)
            o = pl.multiple_of(o - bl * bit, 2 * bl)

        for rdma in pending:
            rdma.wait_send()

    out = pl.pallas_call(
        body,
        out_shape=jax.ShapeDtypeStruct((T, D), jnp.float32),
        in_specs=[
            pl.BlockSpec(memory_space=pltpu.VMEM),
            pl.BlockSpec(memory_space=pltpu.VMEM),
            pl.BlockSpec(memory_space=pl.ANY),
            pl.BlockSpec(memory_space=pl.ANY),
            pl.BlockSpec(memory_space=pltpu.VMEM),
        ],
        out_specs=pl.BlockSpec(memory_space=pltpu.VMEM),
        scratch_shapes=[
            pltpu.VMEM((B, Hloc, Skv, Dh), jnp.float32),
            pltpu.VMEM((B, Hloc, Skv, Dh), jnp.float32),
            pltpu.VMEM((T // 2, D), jnp.bfloat16),
            pltpu.VMEM((T // 4, D), jnp.bfloat16),
            pltpu.VMEM((T // 4, D), jnp.bfloat16),
            pltpu.VMEM((T // 4, D), jnp.bfloat16),
            pltpu.VMEM((T // 4, D), jnp.bfloat16),
            pltpu.VMEM((T // 2, D), jnp.bfloat16),
            pltpu.VMEM((T // 2, D), jnp.bfloat16),
            pltpu.VMEM((T // 4, D), jnp.bfloat16),
            pltpu.VMEM((T // 4, D), jnp.bfloat16),
            pltpu.VMEM((T // 4, D), jnp.bfloat16),
            pltpu.VMEM((T // 4, D), jnp.bfloat16),
            pltpu.VMEM((T // 2, D), jnp.bfloat16),
            pltpu.SemaphoreType.DMA((2 * B * Hloc,)),
            pltpu.SemaphoreType.DMA((6,)),
            pltpu.SemaphoreType.DMA((6,)),
        ],
        compiler_params=pltpu.CompilerParams(collective_id=0),
    )(x, Wq, K_ext, V_ext, Wo)
    return out.reshape(B, Sq, D)


# device time: 36442 ns/iter; 1.7395x vs baseline; 1.7395x over previous
import jax
import jax.numpy as jnp
from jax import lax
from jax.experimental import pallas as pl
from jax.experimental.pallas import tpu as pltpu

N_DEV = 16
MASKS = [1, 2, 4, 8]


def kernel(x, Wq, K_ext, V_ext, Wo):
    B, Sq, D = x.shape
    _, Hd = Wq.shape
    _, Skv, Hq, Dh = K_ext.shape
    Hloc = Hd // Dh
    T = B * Sq

    my = lax.axis_index("i")
    K_loc = jnp.moveaxis(
        lax.dynamic_slice_in_dim(K_ext, my * Hloc, Hloc, axis=2), 2, 1)
    V_loc = jnp.moveaxis(
        lax.dynamic_slice_in_dim(V_ext, my * Hloc, Hloc, axis=2), 2, 1)

    def body(x_ref, wq_ref, k_ref, v_ref, wo_ref, acc_ref,
             sb0, sb1, xs0, xs1, as0, as1,
             rb0, rb1, xr0, xr1, ar0, ar1,
             send_sems, recv_sems):
        my_pos = lax.axis_index("i")
        pending = []

        barrier_sem = pltpu.get_barrier_semaphore()
        for m in MASKS:
            pl.semaphore_signal(
                barrier_sem, inc=1,
                device_id=(jnp.bitwise_xor(my_pos, m),),
                device_id_type=pl.DeviceIdType.MESH)

        qi = lax.broadcasted_iota(jnp.int32, (Sq, Skv), 0)
        ki = lax.broadcasted_iota(jnp.int32, (Sq, Skv), 1)
        mask = (jnp.abs(qi - ki) <= 128) | (ki < 32) | (qi < 32)

        def slab(b):
            qfull = jnp.dot(x_ref[b], wq_ref[...],
                            preferred_element_type=jnp.float32)
            acc = jnp.zeros((Sq, D), jnp.float32)
            for h in range(Hloc):
                q = qfull[:, h * Dh:(h + 1) * Dh]
                k = k_ref[b, h]
                v = v_ref[b, h]
                s = lax.dot_general(
                    q, k, (((1,), (1,)), ((), ())),
                    preferred_element_type=jnp.float32) * 0.125
                s = jnp.where(mask, s, -1e9)
                mx = jnp.max(s, axis=1, keepdims=True)
                w = jnp.exp(s - mx)
                w = w / jnp.sum(w, axis=1, keepdims=True)
                ctx = jnp.dot(w, v, preferred_element_type=jnp.float32)
                acc = acc + jnp.dot(
                    ctx, wo_ref[h * Dh:(h + 1) * Dh, :],
                    preferred_element_type=jnp.float32)
            acc_ref[b * Sq:(b + 1) * Sq, :] = acc

        def exchange(src_buf, dst_buf, m, slot):
            rdma = pltpu.make_async_remote_copy(
                src_ref=src_buf, dst_ref=dst_buf,
                send_sem=send_sems.at[slot], recv_sem=recv_sems.at[slot],
                device_id=(jnp.bitwise_xor(my_pos, m),),
                device_id_type=pl.DeviceIdType.MESH,
            )
            pending.append(rdma)
            return rdma

        hl = T // 2
        bit0 = (jnp.bitwise_and(my_pos, 1) > 0).astype(jnp.int32)
        rdma0 = exchange(sb0, rb0, 1, 0)

        slab(0)
        pl.semaphore_wait(barrier_sem, len(MASKS))

        @pl.when(bit0 == 1)
        def _():
            sb0[...] = acc_ref[0:hl, :].astype(jnp.bfloat16)
            rdma0.start()

        slab(1)

        @pl.when(bit0 == 0)
        def _():
            sb0[...] = acc_ref[hl:T, :].astype(jnp.bfloat16)
            rdma0.start()

        rdma0.wait_recv()
        o = pl.multiple_of(hl * bit0, hl)
        acc_ref[pl.ds(o, hl), :] = (
            acc_ref[pl.ds(o, hl), :] + rb0[...].astype(jnp.float32))

        hl = T // 4
        bit = (jnp.bitwise_and(my_pos, 4) > 0).astype(jnp.int32)
        send_off = pl.multiple_of(o + hl * (1 - bit), hl)
        keep_off = pl.multiple_of(o + hl * bit, hl)
        sb1[...] = acc_ref[pl.ds(send_off, hl), :].astype(jnp.bfloat16)
        rdma = exchange(sb1, rb1, 4, 1)
        rdma.start()
        rdma.wait_recv()
        acc_ref[pl.ds(keep_off, hl), :] = (
            acc_ref[pl.ds(keep_off, hl), :] + rb1[...].astype(jnp.float32))
        o = keep_off

        for slot, (m, sbuf, rbuf) in enumerate(
                [(2, xs0, xr0), (8, xs1, xr1)], start=2):
            sbuf[...] = acc_ref[pl.ds(o, hl), :].astype(jnp.bfloat16)
            rdma = exchange(sbuf, rbuf, m, slot)
            rdma.start()
            rdma.wait_recv()
            acc_ref[pl.ds(o, hl), :] = (
                acc_ref[pl.ds(o, hl), :] + rbuf[...].astype(jnp.float32))

        for slot, (m, bl, sbuf, rbuf) in enumerate(
                [(4, T // 4, as0, ar0), (1, T // 2, as1, ar1)], start=4):
            bit = (jnp.bitwise_and(my_pos, m) > 0).astype(jnp.int32)
            sbuf[...] = acc_ref[pl.ds(o, bl), :].astype(jnp.bfloat16)
            rdma = exchange(sbuf, rbuf, m, slot)
            rdma.start()
            rdma.wait_recv()
            recv_off = pl.multiple_of(o + bl * (1 - 2 * bit), bl)
            acc_ref[pl.ds(recv_off, bl), :] = rbuf[...].astype(jnp.float32)
            o = pl.multiple_of(o - bl * bit, 2 * bl)

        for rdma in pending:
            rdma.wait_send()

    out = pl.pallas_call(
        body,
        out_shape=jax.ShapeDtypeStruct((T, D), jnp.float32),
        in_specs=[pl.BlockSpec(memory_space=pltpu.VMEM)] * 5,
        out_specs=pl.BlockSpec(memory_space=pltpu.VMEM),
        scratch_shapes=[
            pltpu.VMEM((T // 2, D), jnp.bfloat16),
            pltpu.VMEM((T // 4, D), jnp.bfloat16),
            pltpu.VMEM((T // 4, D), jnp.bfloat16),
            pltpu.VMEM((T // 4, D), jnp.bfloat16),
            pltpu.VMEM((T // 4, D), jnp.bfloat16),
            pltpu.VMEM((T // 2, D), jnp.bfloat16),
            pltpu.VMEM((T // 2, D), jnp.bfloat16),
            pltpu.VMEM((T // 4, D), jnp.bfloat16),
            pltpu.VMEM((T // 4, D), jnp.bfloat16),
            pltpu.VMEM((T // 4, D), jnp.bfloat16),
            pltpu.VMEM((T // 4, D), jnp.bfloat16),
            pltpu.VMEM((T // 2, D), jnp.bfloat16),
            pltpu.SemaphoreType.DMA((6,)),
            pltpu.SemaphoreType.DMA((6,)),
        ],
        compiler_params=pltpu.CompilerParams(collective_id=0),
    )(x, Wq, K_loc, V_loc, Wo)
    return out.reshape(B, Sq, D)


# device time: 32910 ns/iter; 1.9262x vs baseline; 1.1073x over previous
import jax
import jax.numpy as jnp
from jax import lax
from jax.experimental import pallas as pl
from jax.experimental.pallas import tpu as pltpu

N_DEV = 16
MASKS = [1, 2, 4, 8]


def kernel(x, Wq, K_ext, V_ext, Wo):
    B, Sq, D = x.shape
    _, Hd = Wq.shape
    _, Skv, Hq, Dh = K_ext.shape
    Hloc = Hd // Dh
    T = B * Sq

    my = lax.axis_index("i")
    K_loc = lax.dynamic_slice_in_dim(K_ext, my * Hloc, Hloc, axis=2)
    K_loc = K_loc.reshape(B, Skv, Hloc * Dh)
    V_loc = lax.dynamic_slice_in_dim(V_ext, my * Hloc, Hloc, axis=2)
    V_loc = V_loc.reshape(B, Skv, Hloc * Dh)

    def body(x_ref, wq_ref, k_ref, v_ref, wo_ref, acc_ref,
             sb0, sb1, xs0, xs1, as0, as1a, as1b,
             rb0, rb1, xr0, xr1, ar0, ar1a, ar1b,
             send_sems, recv_sems):
        my_pos = lax.axis_index("i")
        pending = []

        barrier_sem = pltpu.get_barrier_semaphore()
        for m in MASKS:
            pl.semaphore_signal(
                barrier_sem, inc=1,
                device_id=(jnp.bitwise_xor(my_pos, m),),
                device_id_type=pl.DeviceIdType.MESH)

        qi = lax.broadcasted_iota(jnp.int32, (Sq, Skv), 0)
        ki = lax.broadcasted_iota(jnp.int32, (Sq, Skv), 1)
        mask = (jnp.abs(qi - ki) <= 128) | (ki < 32) | (qi < 32)

        def slab(b):
            qfull = jnp.dot(x_ref[b], wq_ref[...],
                            preferred_element_type=jnp.float32)
            acc = jnp.zeros((Sq, D), jnp.float32)
            for h in range(Hloc):
                q = qfull[:, h * Dh:(h + 1) * Dh]
                k = k_ref[b][:, h * Dh:(h + 1) * Dh]
                v = v_ref[b][:, h * Dh:(h + 1) * Dh]
                s = lax.dot_general(
                    q, k, (((1,), (1,)), ((), ())),
                    preferred_element_type=jnp.float32) * 0.125
                s = jnp.where(mask, s, -1e9)
                mx = jnp.max(s, axis=1, keepdims=True)
                w = jnp.exp(s - mx)
                w = w / jnp.sum(w, axis=1, keepdims=True)
                ctx = jnp.dot(w, v, preferred_element_type=jnp.float32)
                acc = acc + jnp.dot(
                    ctx, wo_ref[h * Dh:(h + 1) * Dh, :],
                    preferred_element_type=jnp.float32)
            acc_ref[b * Sq:(b + 1) * Sq, :] = acc

        def exchange(src_buf, dst_buf, m, slot):
            rdma = pltpu.make_async_remote_copy(
                src_ref=src_buf, dst_ref=dst_buf,
                send_sem=send_sems.at[slot], recv_sem=recv_sems.at[slot],
                device_id=(jnp.bitwise_xor(my_pos, m),),
                device_id_type=pl.DeviceIdType.MESH,
            )
            pending.append(rdma)
            return rdma

        hl = T // 2
        bit0 = (jnp.bitwise_and(my_pos, 1) > 0).astype(jnp.int32)
        rdma0 = exchange(sb0, rb0, 1, 0)

        @pl.when(bit0 == 1)
        def _():
            slab(0)
            pl.semaphore_wait(barrier_sem, len(MASKS))
            sb0[...] = acc_ref[0:hl, :].astype(jnp.bfloat16)
            rdma0.start()
            slab(1)

        @pl.when(bit0 == 0)
        def _():
            slab(1)
            pl.semaphore_wait(barrier_sem, len(MASKS))
            sb0[...] = acc_ref[hl:T, :].astype(jnp.bfloat16)
            rdma0.start()
            slab(0)

        rdma0.wait_recv()
        o = pl.multiple_of(hl * bit0, hl)
        acc_ref[pl.ds(o, hl), :] = (
            acc_ref[pl.ds(o, hl), :] + rb0[...].astype(jnp.float32))

        hl = T // 4
        bit4 = (jnp.bitwise_and(my_pos, 4) > 0).astype(jnp.int32)
        send_off = pl.multiple_of(o + hl * (1 - bit4), hl)
        keep_off = pl.multiple_of(o + hl * bit4, hl)
        sb1[...] = acc_ref[pl.ds(send_off, hl), :].astype(jnp.bfloat16)
        rdma = exchange(sb1, rb1, 4, 1)
        rdma.start()
        rdma.wait_recv()
        acc_ref[pl.ds(keep_off, hl), :] = (
            acc_ref[pl.ds(keep_off, hl), :] + rb1[...].astype(jnp.float32))
        o = keep_off

        for slot, (m, sbuf, rbuf) in enumerate(
                [(2, xs0, xr0), (8, xs1, xr1)], start=2):
            sbuf[...] = acc_ref[pl.ds(o, hl), :].astype(jnp.bfloat16)
            rdma = exchange(sbuf, rbuf, m, slot)
            rdma.start()
            rdma.wait_recv()
            acc_ref[pl.ds(o, hl), :] = (
                acc_ref[pl.ds(o, hl), :] + rbuf[...].astype(jnp.float32))

        slab_base = pl.multiple_of((T // 2) * bit0, T // 2)
        r4_recv_off = pl.multiple_of(slab_base + hl * (1 - bit4), hl)

        as0[...] = acc_ref[pl.ds(o, hl), :].astype(jnp.bfloat16)
        rdma4 = exchange(as0, ar0, 4, 4)
        rdma4.start()

        as1a[...] = acc_ref[pl.ds(o, hl), :].astype(jnp.bfloat16)
        rdma5a = exchange(as1a, ar1a, 1, 5)
        rdma5a.start()

        rdma4.wait_recv()
        acc_ref[pl.ds(r4_recv_off, hl), :] = ar0[...].astype(jnp.float32)

        as1b[...] = acc_ref[pl.ds(r4_recv_off, hl), :].astype(jnp.bfloat16)
        rdma5b = exchange(as1b, ar1b, 1, 6)
        rdma5b.start()

        sib_base = pl.multiple_of((T // 2) * (1 - bit0), T // 2)
        recv_a = pl.multiple_of(sib_base + hl * bit4, hl)
        recv_b = pl.multiple_of(sib_base + hl * (1 - bit4), hl)
        rdma5a.wait_recv()
        acc_ref[pl.ds(recv_a, hl), :] = ar1a[...].astype(jnp.float32)
        rdma5b.wait_recv()
        acc_ref[pl.ds(recv_b, hl), :] = ar1b[...].astype(jnp.float32)

        for rdma in pending:
            rdma.wait_send()

    out = pl.pallas_call(
        body,
        out_shape=jax.ShapeDtypeStruct((T, D), jnp.float32),
        in_specs=[pl.BlockSpec(memory_space=pltpu.VMEM)] * 5,
        out_specs=pl.BlockSpec(memory_space=pltpu.VMEM),
        scratch_shapes=[
            pltpu.VMEM((T // 2, D), jnp.bfloat16),
            pltpu.VMEM((T // 4, D), jnp.bfloat16),
            pltpu.VMEM((T // 4, D), jnp.bfloat16),
            pltpu.VMEM((T // 4, D), jnp.bfloat16),
            pltpu.VMEM((T // 4, D), jnp.bfloat16),
            pltpu.VMEM((T // 4, D), jnp.bfloat16),
            pltpu.VMEM((T // 4, D), jnp.bfloat16),
            pltpu.VMEM((T // 2, D), jnp.bfloat16),
            pltpu.VMEM((T // 4, D), jnp.bfloat16),
            pltpu.VMEM((T // 4, D), jnp.bfloat16),
            pltpu.VMEM((T // 4, D), jnp.bfloat16),
            pltpu.VMEM((T // 4, D), jnp.bfloat16),
            pltpu.VMEM((T // 4, D), jnp.bfloat16),
            pltpu.VMEM((T // 4, D), jnp.bfloat16),
            pltpu.SemaphoreType.DMA((7,)),
            pltpu.SemaphoreType.DMA((7,)),
        ],
        compiler_params=pltpu.CompilerParams(collective_id=0),
    )(x, Wq, K_loc, V_loc, Wo)
    return out.reshape(B, Sq, D)


# device time: 30857 ns/iter; 2.0544x vs baseline; 1.0665x over previous
import jax
import jax.numpy as jnp
from jax import lax
from jax.experimental import pallas as pl
from jax.experimental.pallas import tpu as pltpu

N_DEV = 16
MASKS = [1, 2, 4, 8]


def kernel(x, Wq, K_ext, V_ext, Wo):
    B, Sq, D = x.shape
    _, Hd = Wq.shape
    _, Skv, Hq, Dh = K_ext.shape
    Hloc = Hd // Dh
    T = B * Sq
    HL = T // 4
    HC = D // 2

    my = lax.axis_index("i")
    K_loc = lax.dynamic_slice_in_dim(K_ext, my * Hloc, Hloc, axis=2)
    K_loc = K_loc.reshape(B, Skv, Hloc * Dh)
    V_loc = lax.dynamic_slice_in_dim(V_ext, my * Hloc, Hloc, axis=2)
    V_loc = V_loc.reshape(B, Skv, Hloc * Dh)

    def body(x_ref, wq_ref, k_ref, v_ref, wo_ref, acc_ref,
             sb0, rb0, ss, sr, send_sems, recv_sems):
        my_pos = lax.axis_index("i")
        pending = []

        barrier_sem = pltpu.get_barrier_semaphore()
        for m in MASKS:
            pl.semaphore_signal(
                barrier_sem, inc=1,
                device_id=(jnp.bitwise_xor(my_pos, m),),
                device_id_type=pl.DeviceIdType.MESH)

        qi = lax.broadcasted_iota(jnp.int32, (Sq, Skv), 0)
        ki = lax.broadcasted_iota(jnp.int32, (Sq, Skv), 1)
        mask = (jnp.abs(qi - ki) <= 128) | (ki < 32) | (qi < 32)

        def slab(b):
            qfull = jnp.dot(x_ref[b], wq_ref[...],
                            preferred_element_type=jnp.float32)
            acc = jnp.zeros((Sq, D), jnp.float32)
            for h in range(Hloc):
                q = qfull[:, h * Dh:(h + 1) * Dh]
                k = k_ref[b][:, h * Dh:(h + 1) * Dh]
                v = v_ref[b][:, h * Dh:(h + 1) * Dh]
                s = lax.dot_general(
                    q, k, (((1,), (1,)), ((), ())),
                    preferred_element_type=jnp.float32) * 0.125
                s = jnp.where(mask, s, -1e9)
                mx = jnp.max(s, axis=1, keepdims=True)
                w = jnp.exp(s - mx)
                w = w / jnp.sum(w, axis=1, keepdims=True)
                ctx = jnp.dot(w, v, preferred_element_type=jnp.float32)
                acc = acc + jnp.dot(
                    ctx, wo_ref[h * Dh:(h + 1) * Dh, :],
                    preferred_element_type=jnp.float32)
            acc_ref[b * Sq:(b + 1) * Sq, :] = acc

        hl0 = T // 2
        bit0 = (jnp.bitwise_and(my_pos, 1) > 0).astype(jnp.int32)
        bit4 = (jnp.bitwise_and(my_pos, 4) > 0).astype(jnp.int32)
        rdma0 = pltpu.make_async_remote_copy(
            src_ref=sb0, dst_ref=rb0,
            send_sem=send_sems.at[0], recv_sem=recv_sems.at[0],
            device_id=(jnp.bitwise_xor(my_pos, 1),),
            device_id_type=pl.DeviceIdType.MESH,
        )
        pending.append(rdma0)

        @pl.when(bit0 == 1)
        def _():
            slab(0)
            pl.semaphore_wait(barrier_sem, len(MASKS))
            sb0[...] = acc_ref[0:hl0, :].astype(jnp.bfloat16)
            rdma0.start()
            slab(1)

        @pl.when(bit0 == 0)
        def _():
            slab(1)
            pl.semaphore_wait(barrier_sem, len(MASKS))
            sb0[...] = acc_ref[hl0:T, :].astype(jnp.bfloat16)
            rdma0.start()
            slab(0)

        rdma0.wait_recv()
        slab_base = pl.multiple_of(hl0 * bit0, hl0)
        acc_ref[pl.ds(slab_base, hl0), :] = (
            acc_ref[pl.ds(slab_base, hl0), :] + rb0[...].astype(jnp.float32))

        o1_send = pl.multiple_of(slab_base + HL * (1 - bit4), HL)
        o = pl.multiple_of(slab_base + HL * bit4, HL)
        r4_recv = o1_send
        sib_base = pl.multiple_of(hl0 * (1 - bit0), hl0)
        recv_a = pl.multiple_of(sib_base + HL * bit4, HL)
        recv_b = pl.multiple_of(sib_base + HL * (1 - bit4), HL)

        def cs(idx, rows, c0, m):
            ss[idx] = acc_ref[pl.ds(rows, HL), c0:c0 + HC].astype(
                jnp.bfloat16)
            rdma = pltpu.make_async_remote_copy(
                src_ref=ss.at[idx], dst_ref=sr.at[idx],
                send_sem=send_sems.at[idx + 1],
                recv_sem=recv_sems.at[idx + 1],
                device_id=(jnp.bitwise_xor(my_pos, m),),
                device_id_type=pl.DeviceIdType.MESH,
            )
            rdma.start()
            pending.append(rdma)
            return rdma

        def add(idx, rows, c0, rdma):
            rdma.wait_recv()
            acc_ref[pl.ds(rows, HL), c0:c0 + HC] = (
                acc_ref[pl.ds(rows, HL), c0:c0 + HC]
                + sr[idx].astype(jnp.float32))

        def store(idx, rows, c0, rdma):
            rdma.wait_recv()
            acc_ref[pl.ds(rows, HL), c0:c0 + HC] = (
                sr[idx].astype(jnp.float32))

        A, Bc = 0, HC
        r1a, r1b, x2a, x2b, x3a, x3b = 0, 1, 2, 3, 4, 5
        r4a, r4b, r5aa, r5ab, r5ba, r5bb = 6, 7, 8, 9, 10, 11

        d_r1a = cs(r1a, o1_send, A, 4)
        d_r1b = cs(r1b, o1_send, Bc, 4)
        add(r1a, o, A, d_r1a)
        d_x2a = cs(x2a, o, A, 2)
        add(r1b, o, Bc, d_r1b)
        d_x2b = cs(x2b, o, Bc, 2)
        add(x2a, o, A, d_x2a)
        d_x3a = cs(x3a, o, A, 8)
        add(x2b, o, Bc, d_x2b)
        d_x3b = cs(x3b, o, Bc, 8)
        add(x3a, o, A, d_x3a)
        d_r4a = cs(r4a, o, A, 4)
        d_r5aa = cs(r5aa, o, A, 1)
        add(x3b, o, Bc, d_x3b)
        d_r4b = cs(r4b, o, Bc, 4)
        d_r5ab = cs(r5ab, o, Bc, 1)
        store(r4a, r4_recv, A, d_r4a)
        d_r5ba = cs(r5ba, r4_recv, A, 1)
        store(r4b, r4_recv, Bc, d_r4b)
        d_r5bb = cs(r5bb, r4_recv, Bc, 1)
        store(r5aa, recv_a, A, d_r5aa)
        store(r5ab, recv_a, Bc, d_r5ab)
        store(r5ba, recv_b, A, d_r5ba)
        store(r5bb, recv_b, Bc, d_r5bb)

        for rdma in pending:
            rdma.wait_send()

    out = pl.pallas_call(
        body,
        out_shape=jax.ShapeDtypeStruct((T, D), jnp.float32),
        in_specs=[pl.BlockSpec(memory_space=pltpu.VMEM)] * 5,
        out_specs=pl.BlockSpec(memory_space=pltpu.VMEM),
        scratch_shapes=[
            pltpu.VMEM((T // 2, D), jnp.bfloat16),
            pltpu.VMEM((T // 2, D), jnp.bfloat16),
            pltpu.VMEM((12, HL, HC), jnp.bfloat16),
            pltpu.VMEM((12, HL, HC), jnp.bfloat16),
            pltpu.SemaphoreType.DMA((13,)),
            pltpu.SemaphoreType.DMA((13,)),
        ],
        compiler_params=pltpu.CompilerParams(collective_id=0),
    )(x, Wq, K_loc, V_loc, Wo)
    return out.reshape(B, Sq, D)
